# Initial kernel scaffold; baseline (speedup 1.0000x reference)
import jax
import jax.numpy as jnp
from jax.experimental import pallas as pl


def kernel(pos1, pos2):
    raise NotImplementedError("write your pallas kernel here")


# speedup vs baseline: 3.2843x; 3.2843x over previous
import jax
import jax.numpy as jnp
from jax.experimental import pallas as pl

_N = 8192
_M = 8192
_BN = 512
_BM = 2048


def _psl_kernel(p1_ref, p2t_ref, out_ref):
    x1 = p1_ref[:, 0:1]
    y1 = p1_ref[:, 1:2]
    m = None
    for j in range(_M // _BM):
        x2 = p2t_ref[0:1, j * _BM:(j + 1) * _BM]
        y2 = p2t_ref[1:2, j * _BM:(j + 1) * _BM]
        dx = x1 - x2
        dy = y1 - y2
        d2 = dx * dx + dy * dy
        cm = jnp.min(d2, axis=1, keepdims=True)
        m = cm if m is None else jnp.minimum(m, cm)
    s = jnp.sum(jnp.sqrt(m), keepdims=True).reshape(1, 1) * (1.0 / _N)

    @pl.when(pl.program_id(0) == 0)
    def _init():
        out_ref[:, :] = jnp.zeros((1, 1), jnp.float32)

    out_ref[:, :] += s


def kernel(pos1, pos2):
    pos2t = pos2.T
    out = pl.pallas_call(
        _psl_kernel,
        grid=(_N // _BN,),
        in_specs=[
            pl.BlockSpec((_BN, 2), lambda i: (i, 0)),
            pl.BlockSpec((2, _M), lambda i: (0, 0)),
        ],
        out_specs=pl.BlockSpec((1, 1), lambda i: (0, 0)),
        out_shape=jax.ShapeDtypeStruct((1, 1), jnp.float32),
    )(pos1, pos2t)
    return out[0, 0]


# speedup vs baseline: 3.4916x; 1.0631x over previous
import jax
import jax.numpy as jnp
from jax.experimental import pallas as pl

_N = 8192
_M = 8192
_BN = 512
_BM = 2048


def _psl_kernel(p1_ref, aux_ref, out_ref):
    x1 = p1_ref[:, 0:1]
    y1 = p1_ref[:, 1:2]
    n1 = x1 * x1 + y1 * y1
    m = None
    for j in range(_M // _BM):
        lo, hi = j * _BM, (j + 1) * _BM
        xs = aux_ref[0:1, lo:hi]
        ys = aux_ref[1:2, lo:hi]
        b2 = aux_ref[2:3, lo:hi]
        t = x1 * xs + (y1 * ys + b2)
        cm = jnp.min(t, axis=1, keepdims=True)
        m = cm if m is None else jnp.minimum(m, cm)
    d2 = jnp.maximum(m + n1, 0.0)
    s = jnp.sum(jnp.sqrt(d2), keepdims=True).reshape(1, 1) * (1.0 / _N)

    @pl.when(pl.program_id(0) == 0)
    def _init():
        out_ref[:, :] = jnp.zeros((1, 1), jnp.float32)

    out_ref[:, :] += s


def kernel(pos1, pos2):
    x2 = pos2[:, 0]
    y2 = pos2[:, 1]
    aux = jnp.stack([-2.0 * x2, -2.0 * y2, x2 * x2 + y2 * y2])
    out = pl.pallas_call(
        _psl_kernel,
        grid=(_N // _BN,),
        in_specs=[
            pl.BlockSpec((_BN, 2), lambda i: (i, 0)),
            pl.BlockSpec((3, _M), lambda i: (0, 0)),
        ],
        out_specs=pl.BlockSpec((1, 1), lambda i: (0, 0)),
        out_shape=jax.ShapeDtypeStruct((1, 1), jnp.float32),
    )(pos1, aux)
    return out[0, 0]


# speedup vs baseline: 4.3698x; 1.2515x over previous
import jax
import jax.numpy as jnp
from jax.experimental import pallas as pl

_N = 8192
_M = 8192
_BN = 512
_BM = 2048


def _psl_kernel(p1_ref, p2t_ref, b2_ref, out_ref):
    p1 = p1_ref[:, :]
    x1 = p1[:, 0:1]
    y1 = p1[:, 1:2]
    n1 = x1 * x1 + y1 * y1
    p1m2 = -2.0 * p1
    m = None
    for j in range(_M // _BM):
        lo, hi = j * _BM, (j + 1) * _BM
        g = jax.lax.dot_general(
            p1m2, p2t_ref[:, lo:hi],
            dimension_numbers=(((1,), (0,)), ((), ())),
            preferred_element_type=jnp.float32,
        )
        t = g + b2_ref[0:1, lo:hi]
        cm = jnp.min(t, axis=1, keepdims=True)
        m = cm if m is None else jnp.minimum(m, cm)
    d2 = jnp.maximum(m + n1, 0.0)
    s = jnp.sum(jnp.sqrt(d2), keepdims=True).reshape(1, 1) * (1.0 / _N)

    @pl.when(pl.program_id(0) == 0)
    def _init():
        out_ref[:, :] = jnp.zeros((1, 1), jnp.float32)

    out_ref[:, :] += s


def kernel(pos1, pos2):
    x2 = pos2[:, 0]
    y2 = pos2[:, 1]
    p2t = pos2.T
    b2 = (x2 * x2 + y2 * y2)[None, :]
    out = pl.pallas_call(
        _psl_kernel,
        grid=(_N // _BN,),
        in_specs=[
            pl.BlockSpec((_BN, 2), lambda i: (i, 0)),
            pl.BlockSpec((2, _M), lambda i: (0, 0)),
            pl.BlockSpec((1, _M), lambda i: (0, 0)),
        ],
        out_specs=pl.BlockSpec((1, 1), lambda i: (0, 0)),
        out_shape=jax.ShapeDtypeStruct((1, 1), jnp.float32),
    )(pos1, p2t, b2)
    return out[0, 0]
